# Initial kernel scaffold; baseline (speedup 1.0000x reference)
import jax
import jax.numpy as jnp
from jax.experimental import pallas as pl


def kernel(x, A_tilde, s1_sct, s2_sct, s3_sct, W1, a1, W2, a2):
    raise NotImplementedError("write your pallas kernel here")


# speedup vs baseline: 1.0214x; 1.0214x over previous
import functools

import jax
import jax.numpy as jnp
from jax.experimental import pallas as pl
from jax.experimental.pallas import tpu as pltpu


def _layer_body(h_ref, A_ref, s1_ref, s2_ref, s3_ref, W_ref, a_ref,
                out_ref, hp_ref, *, final):
    @pl.when(pl.program_id(0) == 0)
    def _project():
        hp_ref[...] = jnp.dot(h_ref[...], W_ref[...],
                              preferred_element_type=jnp.float32)

    hp = hp_ref[...]
    a = a_ref[...]
    ps = [jnp.dot(m_ref[...], hp, preferred_element_type=jnp.float32)
          for m_ref in (A_ref, s1_ref, s2_ref, s3_ref)]

    cols = [jnp.dot(p, a[:, s:s + 1], preferred_element_type=jnp.float32)
            for s, p in enumerate(ps)]
    scores = jnp.concatenate(cols, axis=1)
    scores = jnp.where(scores >= 0, scores, 0.2 * scores)
    m = jnp.max(scores, axis=1, keepdims=True)
    e = jnp.exp(scores - m)
    alpha = e / jnp.sum(e, axis=1, keepdims=True)

    out = ps[0] * alpha[:, 0:1]
    for s in range(1, 4):
        out = out + ps[s] * alpha[:, s:s + 1]

    out = jnp.maximum(out, 0.0)
    if final:
        mx = jnp.max(out, axis=1, keepdims=True)
        shifted = out - mx
        lse = jnp.log(jnp.sum(jnp.exp(shifted), axis=1, keepdims=True))
        out = shifted - lse
    out_ref[...] = out


def _layer(h, A, s1, s2, s3, W, a, *, final, block_rows):
    N, Fin = h.shape
    Fout = W.shape[1]
    grid = (N // block_rows,)
    mat_spec = pl.BlockSpec((block_rows, N), lambda i: (i, 0))

    def full(shape):
        return pl.BlockSpec(shape, lambda i: (0, 0))

    return pl.pallas_call(
        functools.partial(_layer_body, final=final),
        grid=grid,
        in_specs=[full((N, Fin)), mat_spec, mat_spec, mat_spec, mat_spec,
                  full((Fin, Fout)), full((Fout, 4))],
        out_specs=pl.BlockSpec((block_rows, Fout), lambda i: (i, 0)),
        out_shape=jax.ShapeDtypeStruct((N, Fout), jnp.float32),
        scratch_shapes=[pltpu.VMEM((N, Fout), jnp.float32)],
        compiler_params=pltpu.CompilerParams(
            dimension_semantics=("arbitrary",)),
    )(h, A, s1, s2, s3, W, a)


def kernel(x, A_tilde, s1_sct, s2_sct, s3_sct, W1, a1, W2, a2):
    h1 = _layer(x, A_tilde, s1_sct, s2_sct, s3_sct, W1, a1,
                final=False, block_rows=80)
    return _layer(h1, A_tilde, s1_sct, s2_sct, s3_sct, W2, a2,
                  final=True, block_rows=80)
